# Initial kernel scaffold; baseline (speedup 1.0000x reference)
import jax
import jax.numpy as jnp
from jax.experimental import pallas as pl


def kernel(features, edge_index, W1, att_src1, att_dst1, b1, W2, att_src2, att_dst2, b2, m1_w, m1_b, m2_w, m2_b, g1_w, g1_b, g2_w, g2_b, ln_g, ln_b):
    raise NotImplementedError("write your pallas kernel here")


# speedup vs baseline: 9.2555x; 9.2555x over previous
import functools

import jax
import jax.numpy as jnp
import numpy as np
from jax.experimental import pallas as pl
from jax.experimental.pallas import tpu as pltpu

DIM = 128
HEADS = 8
DH = DIM // HEADS


def _leaky(x, slope):
    return jnp.where(x >= 0, x, slope * x)


def _table_body(x_ref, w_ref, o_ref):
    o_ref[...] = jnp.dot(x_ref[...], w_ref[...], preferred_element_type=jnp.float32)


def _build_table(x, wcat, block=1000):
    n = x.shape[0]
    k = wcat.shape[1]
    return pl.pallas_call(
        _table_body,
        grid=(n // block,),
        in_specs=[
            pl.BlockSpec((block, DIM), lambda i: (i, 0)),
            pl.BlockSpec((DIM, k), lambda i: (0, 0)),
        ],
        out_specs=pl.BlockSpec((block, k), lambda i: (i, 0)),
        out_shape=jax.ShapeDtypeStruct((n, k), jnp.float32),
    )(x, wcat)


def _att_cat(W, att_src, att_dst):
    a = jnp.zeros((DIM, 2 * HEADS), jnp.float32)
    hs = jnp.arange(DIM) // DH
    ds = jnp.arange(DIM) % DH
    a = a.at[jnp.arange(DIM), hs].set(att_src[hs, ds])
    a = a.at[jnp.arange(DIM), HEADS + hs].set(att_dst[hs, ds])
    return jnp.concatenate([W, W @ a], axis=1)


def _gat_layer(x, src, dst, W, att_src, att_dst, b):
    n = x.shape[0]
    wcat = _att_cat(W, att_src, att_dst)
    table = _build_table(x, wcat)
    xw = table[:, :DIM]
    a_s = table[:, DIM:DIM + HEADS]
    a_d = table[:, DIM + HEADS:]
    w = jnp.exp(_leaky(a_s[src] + a_d[dst], 0.2))
    num = jax.ops.segment_sum(
        (xw[src].reshape(-1, HEADS, DH) * w[:, :, None]).reshape(-1, DIM),
        dst, num_segments=n)
    den = jax.ops.segment_sum(w, dst, num_segments=n)
    out = num.reshape(n, HEADS, DH) / den[:, :, None]
    return out.reshape(n, DIM) + b


def kernel(features, edge_index, W1, att_src1, att_dst1, b1, W2, att_src2, att_dst2, b2, m1_w, m1_b, m2_w, m2_b, g1_w, g1_b, g2_w, g2_b, ln_g, ln_b):
    n = features.shape[0]
    loop = jnp.arange(n, dtype=edge_index.dtype)
    src = jnp.concatenate([edge_index[0], loop])
    dst = jnp.concatenate([edge_index[1], loop])

    h = _leaky(_gat_layer(features, src, dst, W1, att_src1, att_dst1, b1), 0.01)
    h = _leaky(_gat_layer(h, src, dst, W2, att_src2, att_dst2, b2), 0.01)
    s = _leaky(h @ m1_w + m1_b, 0.01) @ m2_w + m2_b
    w = jax.nn.softmax(s[:, 0], axis=0)
    agg = jnp.sum(w[:, None] * h, axis=0)
    agg = _leaky(agg @ g1_w + g1_b, 0.01) @ g2_w + g2_b
    mu = jnp.mean(agg)
    var = jnp.mean((agg - mu) ** 2)
    return (agg - mu) / jnp.sqrt(var + 1e-5) * ln_g + ln_b


# speedup vs baseline: 40.6609x; 4.3932x over previous
import functools

import jax
import jax.numpy as jnp
from jax import lax
from jax.experimental import pallas as pl
from jax.experimental.pallas import tpu as pltpu
from jax.experimental.pallas import tpu_sc as plsc

DIM = 128
HEADS = 8
DH = DIM // HEADS
N = 10000
NPAD = 10240
TW = DIM + 2 * HEADS

NC = 2
NS = 16
NW = NC * NS
EB = 128


def _leaky(x, slope):
    return jnp.where(x >= 0, x, slope * x)


def _table_body(x_ref, w_ref, t_ref, ad_ref):
    t = jnp.dot(x_ref[...], w_ref[...], preferred_element_type=jnp.float32)
    t_ref[...] = t
    ad_ref[...] = t[:, DIM:]


def _build_table(x, wcat, block=1280):
    n = x.shape[0]
    return pl.pallas_call(
        _table_body,
        grid=(n // block,),
        in_specs=[
            pl.BlockSpec((block, DIM), lambda i: (i, 0)),
            pl.BlockSpec((DIM, TW), lambda i: (0, 0)),
        ],
        out_specs=[
            pl.BlockSpec((block, TW), lambda i: (i, 0)),
            pl.BlockSpec((block, 2 * HEADS), lambda i: (i, 0)),
        ],
        out_shape=[
            jax.ShapeDtypeStruct((n, TW), jnp.float32),
            jax.ShapeDtypeStruct((n, 2 * HEADS), jnp.float32),
        ],
    )(x, wcat)


def _finalize1_body(acc_ref, b_ref, wcat_ref, t_ref, ad_ref):
    num = acc_ref[0, :, :DIM] + acc_ref[1, :, :DIM]
    den = acc_ref[0, :, DIM:DIM + HEADS] + acc_ref[1, :, DIM:DIM + HEADS]
    den = den + (den == 0).astype(jnp.float32)
    lane = lax.broadcasted_iota(jnp.int32, (HEADS, DIM), 1)
    head = lax.broadcasted_iota(jnp.int32, (HEADS, DIM), 0)
    spread = (lane // DH == head).astype(jnp.float32)
    den128 = jnp.dot(1.0 / den, spread, preferred_element_type=jnp.float32)
    h = _leaky(num * den128 + b_ref[...], 0.01)
    t = jnp.dot(h, wcat_ref[...], preferred_element_type=jnp.float32)
    t_ref[...] = t
    ad_ref[...] = t[:, DIM:]


def _finalize1(acc, b, wcat, block=1280):
    return pl.pallas_call(
        _finalize1_body,
        grid=(NPAD // block,),
        in_specs=[
            pl.BlockSpec((2, block, TW), lambda i: (0, i, 0)),
            pl.BlockSpec((1, DIM), lambda i: (0, 0)),
            pl.BlockSpec((DIM, TW), lambda i: (0, 0)),
        ],
        out_specs=[
            pl.BlockSpec((block, TW), lambda i: (i, 0)),
            pl.BlockSpec((block, 2 * HEADS), lambda i: (i, 0)),
        ],
        out_shape=[
            jax.ShapeDtypeStruct((NPAD, TW), jnp.float32),
            jax.ShapeDtypeStruct((NPAD, 2 * HEADS), jnp.float32),
        ],
    )(acc, b, wcat)


def _finalize2_body(acc_ref, b_ref, m1_ref, m1b_ref, m2_ref, sh_ref, se_ref, *, block):
    i = pl.program_id(0)
    num = acc_ref[0, :, :DIM] + acc_ref[1, :, :DIM]
    den = acc_ref[0, :, DIM:DIM + HEADS] + acc_ref[1, :, DIM:DIM + HEADS]
    den = den + (den == 0).astype(jnp.float32)
    lane = lax.broadcasted_iota(jnp.int32, (HEADS, DIM), 1)
    head = lax.broadcasted_iota(jnp.int32, (HEADS, DIM), 0)
    spread = (lane // DH == head).astype(jnp.float32)
    den128 = jnp.dot(1.0 / den, spread, preferred_element_type=jnp.float32)
    h = _leaky(num * den128 + b_ref[...], 0.01)
    hid = _leaky(jnp.dot(h, m1_ref[...], preferred_element_type=jnp.float32)
                 + m1b_ref[...], 0.01)
    s = jnp.dot(hid, m2_ref[...], preferred_element_type=jnp.float32)
    row = i * block + lax.broadcasted_iota(jnp.int32, (block, 1), 0)
    e = jnp.where(row < N, jnp.exp(s), 0.0)

    @pl.when(i == 0)
    def _():
        sh_ref[...] = jnp.zeros_like(sh_ref)
        se_ref[...] = jnp.zeros_like(se_ref)

    sh_ref[...] += jnp.sum(e * h, axis=0, keepdims=True)
    se_ref[...] += jnp.sum(e, axis=0, keepdims=True)


def _finalize2(acc, b, m1_w, m1_b, m2_w, block=1280):
    return pl.pallas_call(
        functools.partial(_finalize2_body, block=block),
        grid=(NPAD // block,),
        in_specs=[
            pl.BlockSpec((2, block, TW), lambda i: (0, i, 0)),
            pl.BlockSpec((1, DIM), lambda i: (0, 0)),
            pl.BlockSpec((DIM, DIM // 2), lambda i: (0, 0)),
            pl.BlockSpec((1, DIM // 2), lambda i: (0, 0)),
            pl.BlockSpec((DIM // 2, 1), lambda i: (0, 0)),
        ],
        out_specs=[
            pl.BlockSpec((1, DIM), lambda i: (0, 0)),
            pl.BlockSpec((1, 1), lambda i: (0, 0)),
        ],
        out_shape=[
            jax.ShapeDtypeStruct((1, DIM), jnp.float32),
            jax.ShapeDtypeStruct((1, 1), jnp.float32),
        ],
    )(acc, b, m1_w, m1_b, m2_w)


def _head_body(sh_ref, se_ref, g1_ref, g1b_ref, g2_ref, g2b_ref, lng_ref, lnb_ref, o_ref):
    agg = sh_ref[...] / se_ref[0, 0]
    a1 = _leaky(jnp.dot(agg, g1_ref[...], preferred_element_type=jnp.float32)
                + g1b_ref[...], 0.01)
    a2 = jnp.dot(a1, g2_ref[...], preferred_element_type=jnp.float32) + g2b_ref[...]
    mu = jnp.mean(a2)
    var = jnp.mean((a2 - mu) ** 2)
    o_ref[...] = (a2 - mu) / jnp.sqrt(var + 1e-5) * lng_ref[...] + lnb_ref[...]


def _head(sh, se, g1_w, g1_b, g2_w, g2_b, ln_g, ln_b):
    return pl.pallas_call(
        _head_body,
        out_shape=jax.ShapeDtypeStruct((1, DIM), jnp.float32),
    )(sh, se, g1_w, g1_b.reshape(1, -1), g2_w, g2_b.reshape(1, -1),
      ln_g.reshape(1, -1), ln_b.reshape(1, -1))


def _sc_edge_kernel(epad):
    chunk = epad // NW
    nblk = chunk // EB
    mesh = plsc.VectorSubcoreMesh(core_axis_name="c", subcore_axis_name="s",
                                  num_cores=NC, num_subcores=NS)

    @functools.partial(
        pl.kernel,
        out_type=jax.ShapeDtypeStruct((NC, NPAD, TW), jnp.float32),
        mesh=mesh,
        compiler_params=pltpu.CompilerParams(use_tc_tiling_on_sc=False,
                                             needs_layout_passes=False),
        scratch_types=[
            pltpu.VMEM_SHARED((NPAD, TW), jnp.float32),
            pltpu.VMEM((EB,), jnp.int32),
            pltpu.VMEM((EB,), jnp.int32),
            pltpu.VMEM((EB, TW), jnp.float32),
            pltpu.VMEM((EB, 2 * HEADS), jnp.float32),
            pltpu.VMEM((16, EB), jnp.float32),
            pltpu.SemaphoreType.DMA,
            pltpu.SemaphoreType.DMA,
        ],
    )
    def edge_kernel(table_hbm, ad_hbm, src_hbm, dst_hbm, zeros_hbm, out_hbm,
                    acc_sh, src_v, dst_v, rows_v, ad_v, wbuf_v, sem, sem2):
        cid = lax.axis_index("c")
        sid = lax.axis_index("s")
        wid = sid * NC + cid

        zrows = NPAD // NS
        pltpu.sync_copy(zeros_hbm.at[pl.ds(sid * zrows, zrows)],
                        acc_sh.at[pl.ds(sid * zrows, zrows)])
        z16 = jnp.zeros((16,), jnp.float32)
        for r in range(HEADS, 16):
            for c0 in range(0, EB, 16):
                wbuf_v[r, pl.ds(c0, 16)] = z16
        plsc.subcore_barrier()

        lanes = lax.iota(jnp.int32, 16)

        def block_body(blk, carry):
            ebase = wid * chunk + blk * EB
            pltpu.sync_copy(src_hbm.at[pl.ds(ebase, EB)], src_v)
            pltpu.sync_copy(dst_hbm.at[pl.ds(ebase, EB)], dst_v)
            pltpu.async_copy(table_hbm.at[src_v], rows_v, sem).wait()
            pltpu.async_copy(ad_hbm.at[dst_v], ad_v, sem2).wait()

            def group_body(g, c2):
                eidx = lanes + g * 16
                for h in range(HEADS):
                    a_s = plsc.load_gather(
                        rows_v, [eidx, jnp.full((16,), DIM + h, jnp.int32)])
                    a_d = plsc.load_gather(
                        ad_v, [eidx, jnp.full((16,), HEADS + h, jnp.int32)])
                    al = a_s + a_d
                    al = jnp.where(al >= 0, al, 0.2 * al)
                    wbuf_v[h, pl.ds(g * 16, 16)] = jnp.exp(al)
                for k in range(16):
                    e = g * 16 + k
                    efull = jnp.full((16,), e, jnp.int32)
                    denv = plsc.load_gather(wbuf_v, [lanes, efull])
                    for h in range(HEADS):
                        wb = plsc.load_gather(
                            wbuf_v, [jnp.full((16,), h, jnp.int32), efull])
                        rows_v[e, pl.ds(h * DH, 16)] = rows_v[e, pl.ds(h * DH, 16)] * wb
                    rows_v[e, pl.ds(DIM, 16)] = denv
                return c2

            lax.fori_loop(0, EB // 16, group_body, 0)
            pltpu.sync_copy(rows_v, acc_sh.at[dst_v], add=True)
            return carry

        lax.fori_loop(0, nblk, block_body, 0)
        plsc.subcore_barrier()
        pltpu.sync_copy(acc_sh.at[pl.ds(sid * zrows, zrows)],
                        out_hbm.at[cid, pl.ds(sid * zrows, zrows)])

    return edge_kernel


def _att_cat(W, att_src, att_dst):
    a = jnp.zeros((DIM, 2 * HEADS), jnp.float32)
    hs = jnp.arange(DIM) // DH
    ds = jnp.arange(DIM) % DH
    a = a.at[jnp.arange(DIM), hs].set(att_src[hs, ds])
    a = a.at[jnp.arange(DIM), HEADS + hs].set(att_dst[hs, ds])
    return jnp.concatenate([W, W @ a], axis=1)


def kernel(features, edge_index, W1, att_src1, att_dst1, b1, W2, att_src2, att_dst2, b2, m1_w, m1_b, m2_w, m2_b, g1_w, g1_b, g2_w, g2_b, ln_g, ln_b):
    ne = edge_index.shape[1] + N
    epad = ((ne + NW * EB - 1) // (NW * EB)) * (NW * EB)
    loop = jnp.arange(N, dtype=jnp.int32)
    dummy = jnp.full((epad - ne,), N, jnp.int32)
    src = jnp.concatenate([edge_index[0].astype(jnp.int32), loop, dummy])
    dst = jnp.concatenate([edge_index[1].astype(jnp.int32), loop, dummy])
    x = jnp.pad(features, ((0, NPAD - N), (0, 0)))
    zeros_tab = jnp.zeros((NPAD, TW), jnp.float32)

    edge_sc = _sc_edge_kernel(epad)

    wcat1 = _att_cat(W1, att_src1, att_dst1)
    table1, ad1 = _build_table(x, wcat1)
    acc1 = edge_sc(table1, ad1, src, dst, zeros_tab)

    wcat2 = _att_cat(W2, att_src2, att_dst2)
    table2, ad2 = _finalize1(acc1, b1.reshape(1, -1), wcat2)
    acc2 = edge_sc(table2, ad2, src, dst, zeros_tab)

    sh, se = _finalize2(acc2, b2.reshape(1, -1), m1_w, m1_b.reshape(1, -1), m2_w)

    out = _head(sh, se, g1_w, g1_b, g2_w, g2_b, ln_g, ln_b)
    return out.reshape(DIM)


# speedup vs baseline: 80.1581x; 1.9714x over previous
import functools

import jax
import jax.numpy as jnp
from jax import lax
from jax.experimental import pallas as pl
from jax.experimental.pallas import tpu as pltpu
from jax.experimental.pallas import tpu_sc as plsc

DIM = 128
HEADS = 8
DH = DIM // HEADS
N = 10000
NPAD = 10112
TW = DIM + 2 * HEADS

NC = 2
NS = 16
NW = NC * NS
EB = 64


def _leaky(x, slope):
    return jnp.where(x >= 0, x, slope * x)


def _table_body(x_ref, w_ref, t_ref, ad_ref):
    t = jnp.dot(x_ref[...], w_ref[...], preferred_element_type=jnp.float32)
    t_ref[...] = t
    ad_ref[...] = t[:, DIM:]


def _build_table(x, wcat, block=1264):
    n = x.shape[0]
    return pl.pallas_call(
        _table_body,
        grid=(n // block,),
        in_specs=[
            pl.BlockSpec((block, DIM), lambda i: (i, 0)),
            pl.BlockSpec((DIM, TW), lambda i: (0, 0)),
        ],
        out_specs=[
            pl.BlockSpec((block, TW), lambda i: (i, 0)),
            pl.BlockSpec((block, 2 * HEADS), lambda i: (i, 0)),
        ],
        out_shape=[
            jax.ShapeDtypeStruct((n, TW), jnp.float32),
            jax.ShapeDtypeStruct((n, 2 * HEADS), jnp.float32),
        ],
    )(x, wcat)


def _finalize1_body(acc_ref, b_ref, wcat_ref, t_ref, ad_ref):
    num = acc_ref[0, :, :DIM] + acc_ref[1, :, :DIM]
    den = acc_ref[0, :, DIM:DIM + HEADS] + acc_ref[1, :, DIM:DIM + HEADS]
    den = den + (den == 0).astype(jnp.float32)
    lane = lax.broadcasted_iota(jnp.int32, (HEADS, DIM), 1)
    head = lax.broadcasted_iota(jnp.int32, (HEADS, DIM), 0)
    spread = (lane // DH == head).astype(jnp.float32)
    den128 = jnp.dot(1.0 / den, spread, preferred_element_type=jnp.float32)
    h = _leaky(num * den128 + b_ref[...], 0.01)
    t = jnp.dot(h, wcat_ref[...], preferred_element_type=jnp.float32)
    t_ref[...] = t
    ad_ref[...] = t[:, DIM:]


def _finalize1(acc, b, wcat, block=1264):
    return pl.pallas_call(
        _finalize1_body,
        grid=(NPAD // block,),
        in_specs=[
            pl.BlockSpec((2, block, TW), lambda i: (0, i, 0)),
            pl.BlockSpec((1, DIM), lambda i: (0, 0)),
            pl.BlockSpec((DIM, TW), lambda i: (0, 0)),
        ],
        out_specs=[
            pl.BlockSpec((block, TW), lambda i: (i, 0)),
            pl.BlockSpec((block, 2 * HEADS), lambda i: (i, 0)),
        ],
        out_shape=[
            jax.ShapeDtypeStruct((NPAD, TW), jnp.float32),
            jax.ShapeDtypeStruct((NPAD, 2 * HEADS), jnp.float32),
        ],
    )(acc, b, wcat)


def _finalize2_body(acc_ref, b_ref, m1_ref, m1b_ref, m2_ref, sh_ref, se_ref, *, block):
    i = pl.program_id(0)
    num = acc_ref[0, :, :DIM] + acc_ref[1, :, :DIM]
    den = acc_ref[0, :, DIM:DIM + HEADS] + acc_ref[1, :, DIM:DIM + HEADS]
    den = den + (den == 0).astype(jnp.float32)
    lane = lax.broadcasted_iota(jnp.int32, (HEADS, DIM), 1)
    head = lax.broadcasted_iota(jnp.int32, (HEADS, DIM), 0)
    spread = (lane // DH == head).astype(jnp.float32)
    den128 = jnp.dot(1.0 / den, spread, preferred_element_type=jnp.float32)
    h = _leaky(num * den128 + b_ref[...], 0.01)
    hid = _leaky(jnp.dot(h, m1_ref[...], preferred_element_type=jnp.float32)
                 + m1b_ref[...], 0.01)
    s = jnp.dot(hid, m2_ref[...], preferred_element_type=jnp.float32)
    row = i * block + lax.broadcasted_iota(jnp.int32, (block, 1), 0)
    e = jnp.where(row < N, jnp.exp(s), 0.0)

    @pl.when(i == 0)
    def _():
        sh_ref[...] = jnp.zeros_like(sh_ref)
        se_ref[...] = jnp.zeros_like(se_ref)

    sh_ref[...] += jnp.sum(e * h, axis=0, keepdims=True)
    se_ref[...] += jnp.sum(e, axis=0, keepdims=True)


def _finalize2(acc, b, m1_w, m1_b, m2_w, block=1264):
    return pl.pallas_call(
        functools.partial(_finalize2_body, block=block),
        grid=(NPAD // block,),
        in_specs=[
            pl.BlockSpec((2, block, TW), lambda i: (0, i, 0)),
            pl.BlockSpec((1, DIM), lambda i: (0, 0)),
            pl.BlockSpec((DIM, DIM // 2), lambda i: (0, 0)),
            pl.BlockSpec((1, DIM // 2), lambda i: (0, 0)),
            pl.BlockSpec((DIM // 2, 1), lambda i: (0, 0)),
        ],
        out_specs=[
            pl.BlockSpec((1, DIM), lambda i: (0, 0)),
            pl.BlockSpec((1, 1), lambda i: (0, 0)),
        ],
        out_shape=[
            jax.ShapeDtypeStruct((1, DIM), jnp.float32),
            jax.ShapeDtypeStruct((1, 1), jnp.float32),
        ],
    )(acc, b, m1_w, m1_b, m2_w)


def _head_body(sh_ref, se_ref, g1_ref, g1b_ref, g2_ref, g2b_ref, lng_ref, lnb_ref, o_ref):
    agg = sh_ref[...] / se_ref[0, 0]
    a1 = _leaky(jnp.dot(agg, g1_ref[...], preferred_element_type=jnp.float32)
                + g1b_ref[...], 0.01)
    a2 = jnp.dot(a1, g2_ref[...], preferred_element_type=jnp.float32) + g2b_ref[...]
    mu = jnp.mean(a2)
    var = jnp.mean((a2 - mu) ** 2)
    o_ref[...] = (a2 - mu) / jnp.sqrt(var + 1e-5) * lng_ref[...] + lnb_ref[...]


def _head(sh, se, g1_w, g1_b, g2_w, g2_b, ln_g, ln_b):
    return pl.pallas_call(
        _head_body,
        out_shape=jax.ShapeDtypeStruct((1, DIM), jnp.float32),
    )(sh, se, g1_w, g1_b.reshape(1, -1), g2_w, g2_b.reshape(1, -1),
      ln_g.reshape(1, -1), ln_b.reshape(1, -1))


def _bcast_lane(v, k):
    idx = jnp.full((16, 1), k, jnp.int32)
    return lax.gather(
        v, idx,
        lax.GatherDimensionNumbers(offset_dims=(), collapsed_slice_dims=(0,),
                                   start_index_map=(0,)),
        (1,), mode=lax.GatherScatterMode.PROMISE_IN_BOUNDS)


def _sc_edge_kernel(epad):
    chunk = epad // NW
    nblk = chunk // EB
    npairs = nblk // 2
    mesh = plsc.VectorSubcoreMesh(core_axis_name="c", subcore_axis_name="s",
                                  num_cores=NC, num_subcores=NS)

    @functools.partial(
        pl.kernel,
        out_type=jax.ShapeDtypeStruct((NC, NPAD, TW), jnp.float32),
        mesh=mesh,
        compiler_params=pltpu.CompilerParams(use_tc_tiling_on_sc=False,
                                             needs_layout_passes=False),
        scratch_types=[
            pltpu.VMEM_SHARED((NPAD, TW), jnp.float32),
            [pltpu.VMEM((EB,), jnp.int32)] * 2,
            [pltpu.VMEM((EB,), jnp.int32)] * 2,
            [pltpu.VMEM((EB, TW), jnp.float32)] * 2,
            [pltpu.VMEM((EB, 2 * HEADS), jnp.float32)] * 2,
            [pltpu.VMEM((EB, TW), jnp.float32)] * 2,
            [pltpu.VMEM((EB,), jnp.int32)] * 2,
            pltpu.VMEM((HEADS, EB), jnp.float32),
            [pltpu.SemaphoreType.DMA] * 2,
            [pltpu.SemaphoreType.DMA] * 2,
            [pltpu.SemaphoreType.DMA] * 2,
        ],
    )
    def edge_kernel(table_hbm, ad_hbm, src_hbm, dst_hbm, zeros_hbm, out_hbm,
                    acc_sh, src_v, dst_v, rows_v, ad_v, msg_v, sdst_v, wbuf_v,
                    sem_g, sem_a, sem_s):
        cid = lax.axis_index("c")
        sid = lax.axis_index("s")
        wid = sid * NC + cid

        zrows = NPAD // NS
        pltpu.sync_copy(zeros_hbm.at[pl.ds(sid * zrows, zrows)],
                        acc_sh.at[pl.ds(sid * zrows, zrows)])
        plsc.subcore_barrier()

        lanes = lax.iota(jnp.int32, 16)

        def fetch(blk, p):
            ebase = wid * chunk + blk * EB
            pltpu.sync_copy(src_hbm.at[pl.ds(ebase, EB)], src_v[p])
            pltpu.sync_copy(dst_hbm.at[pl.ds(ebase, EB)], dst_v[p])
            pltpu.async_copy(table_hbm.at[src_v[p]], rows_v[p], sem_g[p])
            pltpu.async_copy(ad_hbm.at[dst_v[p]], ad_v[p], sem_a[p])

        def wait_gather(p):
            pltpu.make_async_copy(table_hbm.at[src_v[p]], rows_v[p], sem_g[p]).wait()
            pltpu.make_async_copy(ad_hbm.at[dst_v[p]], ad_v[p], sem_a[p]).wait()

        def wait_scatter(p):
            pltpu.make_async_copy(msg_v[p], acc_sh.at[sdst_v[p]], sem_s[p]).wait()

        def compute(p):
            def group_body(g, c2):
                eidx = lanes + g * 16
                ws = []
                for h in range(HEADS):
                    a_s = plsc.load_gather(
                        rows_v[p], [eidx, jnp.full((16,), DIM + h, jnp.int32)])
                    a_d = plsc.load_gather(
                        ad_v[p], [eidx, jnp.full((16,), HEADS + h, jnp.int32)])
                    al = a_s + a_d
                    al = jnp.where(al >= 0, al, 0.2 * al)
                    w = jnp.exp(al)
                    ws.append(w)
                    wbuf_v[h, pl.ds(g * 16, 16)] = w
                for k in range(16):
                    e = g * 16 + k
                    denv = plsc.load_gather(
                        wbuf_v, [lanes & 7, jnp.full((16,), e, jnp.int32)])
                    for h in range(HEADS):
                        wb = _bcast_lane(ws[h], k)
                        msg_v[p][e, pl.ds(h * DH, 16)] = (
                            rows_v[p][e, pl.ds(h * DH, 16)] * wb)
                    msg_v[p][e, pl.ds(DIM, 16)] = denv
                return c2

            lax.fori_loop(0, EB // 16, group_body, 0)
            for c0 in range(0, EB, 16):
                sdst_v[p][pl.ds(c0, 16)] = dst_v[p][pl.ds(c0, 16)]
            pltpu.async_copy(msg_v[p], acc_sh.at[sdst_v[p]], sem_s[p], add=True)

        fetch(0, 0)

        def pair_body(t, carry):
            fetch(2 * t + 1, 1)
            wait_gather(0)

            @pl.when(t > 0)
            def _():
                wait_scatter(0)
            compute(0)

            @pl.when(t < npairs - 1)
            def _():
                fetch(2 * t + 2, 0)
            wait_gather(1)

            @pl.when(t > 0)
            def _():
                wait_scatter(1)
            compute(1)
            return carry

        lax.fori_loop(0, npairs, pair_body, 0)
        wait_scatter(0)
        wait_scatter(1)

        plsc.subcore_barrier()
        pltpu.sync_copy(acc_sh.at[pl.ds(sid * zrows, zrows)],
                        out_hbm.at[cid, pl.ds(sid * zrows, zrows)])

    return edge_kernel


def _att_cat(W, att_src, att_dst):
    a = jnp.zeros((DIM, 2 * HEADS), jnp.float32)
    hs = jnp.arange(DIM) // DH
    ds = jnp.arange(DIM) % DH
    a = a.at[jnp.arange(DIM), hs].set(att_src[hs, ds])
    a = a.at[jnp.arange(DIM), HEADS + hs].set(att_dst[hs, ds])
    return jnp.concatenate([W, W @ a], axis=1)


def kernel(features, edge_index, W1, att_src1, att_dst1, b1, W2, att_src2, att_dst2, b2, m1_w, m1_b, m2_w, m2_b, g1_w, g1_b, g2_w, g2_b, ln_g, ln_b):
    ne = edge_index.shape[1] + N
    epad = ((ne + NW * EB - 1) // (NW * EB)) * (NW * EB)
    loop = jnp.arange(N, dtype=jnp.int32)
    dummy = jnp.full((epad - ne,), N, jnp.int32)
    src = jnp.concatenate([edge_index[0].astype(jnp.int32), loop, dummy])
    dst = jnp.concatenate([edge_index[1].astype(jnp.int32), loop, dummy])
    x = jnp.pad(features, ((0, NPAD - N), (0, 0)))
    zeros_tab = jnp.zeros((NPAD, TW), jnp.float32)

    edge_sc = _sc_edge_kernel(epad)

    wcat1 = _att_cat(W1, att_src1, att_dst1)
    table1, ad1 = _build_table(x, wcat1)
    acc1 = edge_sc(table1, ad1, src, dst, zeros_tab)

    wcat2 = _att_cat(W2, att_src2, att_dst2)
    table2, ad2 = _finalize1(acc1, b1.reshape(1, -1), wcat2)
    acc2 = edge_sc(table2, ad2, src, dst, zeros_tab)

    sh, se = _finalize2(acc2, b2.reshape(1, -1), m1_w, m1_b.reshape(1, -1), m2_w)

    out = _head(sh, se, g1_w, g1_b, g2_w, g2_b, ln_g, ln_b)
    return out.reshape(DIM)


# speedup vs baseline: 131.9433x; 1.6460x over previous
import functools

import jax
import jax.numpy as jnp
from jax import lax
from jax.experimental import pallas as pl
from jax.experimental.pallas import tpu as pltpu
from jax.experimental.pallas import tpu_sc as plsc

DIM = 128
HEADS = 8
DH = DIM // HEADS
N = 10000
NPAD = 10112
TW = DIM + 2 * HEADS

NC = 2
NS = 16
NW = NC * NS
EB = 64


def _leaky(x, slope):
    return jnp.where(x >= 0, x, slope * x)


def _table_body(x_ref, w_ref, t_ref, ad_ref):
    t = jnp.dot(x_ref[...], w_ref[...], preferred_element_type=jnp.float32)
    t_ref[...] = t
    ad_ref[...] = t[:, DIM:]


def _build_table(x, wcat, block=1264):
    n = x.shape[0]
    return pl.pallas_call(
        _table_body,
        grid=(n // block,),
        in_specs=[
            pl.BlockSpec((block, DIM), lambda i: (i, 0)),
            pl.BlockSpec((DIM, TW), lambda i: (0, 0)),
        ],
        out_specs=[
            pl.BlockSpec((block, TW), lambda i: (i, 0)),
            pl.BlockSpec((block, 2 * HEADS), lambda i: (i, 0)),
        ],
        out_shape=[
            jax.ShapeDtypeStruct((n, TW), jnp.float32),
            jax.ShapeDtypeStruct((n, 2 * HEADS), jnp.float32),
        ],
    )(x, wcat)


def _finalize1_body(acc_ref, b_ref, wcat_ref, t_ref, ad_ref):
    num = acc_ref[0, :, :DIM] + acc_ref[1, :, :DIM]
    den = acc_ref[0, :, DIM:DIM + HEADS] + acc_ref[1, :, DIM:DIM + HEADS]
    den = den + (den == 0).astype(jnp.float32)
    lane = lax.broadcasted_iota(jnp.int32, (HEADS, DIM), 1)
    head = lax.broadcasted_iota(jnp.int32, (HEADS, DIM), 0)
    spread = (lane // DH == head).astype(jnp.float32)
    den128 = jnp.dot(1.0 / den, spread, preferred_element_type=jnp.float32)
    h = _leaky(num * den128 + b_ref[...], 0.01)
    t = jnp.dot(h, wcat_ref[...], preferred_element_type=jnp.float32)
    t_ref[...] = t
    ad_ref[...] = t[:, DIM:]


def _finalize1(acc, b, wcat, block=1264):
    return pl.pallas_call(
        _finalize1_body,
        grid=(NPAD // block,),
        in_specs=[
            pl.BlockSpec((2, block, TW), lambda i: (0, i, 0)),
            pl.BlockSpec((1, DIM), lambda i: (0, 0)),
            pl.BlockSpec((DIM, TW), lambda i: (0, 0)),
        ],
        out_specs=[
            pl.BlockSpec((block, TW), lambda i: (i, 0)),
            pl.BlockSpec((block, 2 * HEADS), lambda i: (i, 0)),
        ],
        out_shape=[
            jax.ShapeDtypeStruct((NPAD, TW), jnp.float32),
            jax.ShapeDtypeStruct((NPAD, 2 * HEADS), jnp.float32),
        ],
    )(acc, b, wcat)


def _finalize2_body(acc_ref, b_ref, m1_ref, m1b_ref, m2_ref, sh_ref, se_ref, *, block):
    i = pl.program_id(0)
    num = acc_ref[0, :, :DIM] + acc_ref[1, :, :DIM]
    den = acc_ref[0, :, DIM:DIM + HEADS] + acc_ref[1, :, DIM:DIM + HEADS]
    den = den + (den == 0).astype(jnp.float32)
    lane = lax.broadcasted_iota(jnp.int32, (HEADS, DIM), 1)
    head = lax.broadcasted_iota(jnp.int32, (HEADS, DIM), 0)
    spread = (lane // DH == head).astype(jnp.float32)
    den128 = jnp.dot(1.0 / den, spread, preferred_element_type=jnp.float32)
    h = _leaky(num * den128 + b_ref[...], 0.01)
    hid = _leaky(jnp.dot(h, m1_ref[...], preferred_element_type=jnp.float32)
                 + m1b_ref[...], 0.01)
    s = jnp.dot(hid, m2_ref[...], preferred_element_type=jnp.float32)
    row = i * block + lax.broadcasted_iota(jnp.int32, (block, 1), 0)
    e = jnp.where(row < N, jnp.exp(s), 0.0)

    @pl.when(i == 0)
    def _():
        sh_ref[...] = jnp.zeros_like(sh_ref)
        se_ref[...] = jnp.zeros_like(se_ref)

    sh_ref[...] += jnp.sum(e * h, axis=0, keepdims=True)
    se_ref[...] += jnp.sum(e, axis=0, keepdims=True)


def _finalize2(acc, b, m1_w, m1_b, m2_w, block=1264):
    return pl.pallas_call(
        functools.partial(_finalize2_body, block=block),
        grid=(NPAD // block,),
        in_specs=[
            pl.BlockSpec((2, block, TW), lambda i: (0, i, 0)),
            pl.BlockSpec((1, DIM), lambda i: (0, 0)),
            pl.BlockSpec((DIM, DIM // 2), lambda i: (0, 0)),
            pl.BlockSpec((1, DIM // 2), lambda i: (0, 0)),
            pl.BlockSpec((DIM // 2, 1), lambda i: (0, 0)),
        ],
        out_specs=[
            pl.BlockSpec((1, DIM), lambda i: (0, 0)),
            pl.BlockSpec((1, 1), lambda i: (0, 0)),
        ],
        out_shape=[
            jax.ShapeDtypeStruct((1, DIM), jnp.float32),
            jax.ShapeDtypeStruct((1, 1), jnp.float32),
        ],
    )(acc, b, m1_w, m1_b, m2_w)


def _head_body(sh_ref, se_ref, g1_ref, g1b_ref, g2_ref, g2b_ref, lng_ref, lnb_ref, o_ref):
    agg = sh_ref[...] / se_ref[0, 0]
    a1 = _leaky(jnp.dot(agg, g1_ref[...], preferred_element_type=jnp.float32)
                + g1b_ref[...], 0.01)
    a2 = jnp.dot(a1, g2_ref[...], preferred_element_type=jnp.float32) + g2b_ref[...]
    mu = jnp.mean(a2)
    var = jnp.mean((a2 - mu) ** 2)
    o_ref[...] = (a2 - mu) / jnp.sqrt(var + 1e-5) * lng_ref[...] + lnb_ref[...]


def _head(sh, se, g1_w, g1_b, g2_w, g2_b, ln_g, ln_b):
    return pl.pallas_call(
        _head_body,
        out_shape=jax.ShapeDtypeStruct((1, DIM), jnp.float32),
    )(sh, se, g1_w, g1_b.reshape(1, -1), g2_w, g2_b.reshape(1, -1),
      ln_g.reshape(1, -1), ln_b.reshape(1, -1))


def _bcast_lane(v, k):
    idx = jnp.full((16, 1), k, jnp.int32)
    return lax.gather(
        v, idx,
        lax.GatherDimensionNumbers(offset_dims=(), collapsed_slice_dims=(0,),
                                   start_index_map=(0,)),
        (1,), mode=lax.GatherScatterMode.PROMISE_IN_BOUNDS)


def _sc_edge_kernel(epad):
    chunk = epad // NW
    nblk = chunk // EB
    npairs = nblk // 2
    mesh = plsc.VectorSubcoreMesh(core_axis_name="c", subcore_axis_name="s",
                                  num_cores=NC, num_subcores=NS)

    @functools.partial(
        pl.kernel,
        out_type=jax.ShapeDtypeStruct((NC, NPAD, TW), jnp.float32),
        mesh=mesh,
        compiler_params=pltpu.CompilerParams(use_tc_tiling_on_sc=False,
                                             needs_layout_passes=False),
        scratch_types=[
            pltpu.VMEM_SHARED((NPAD, TW), jnp.float32),
            [pltpu.VMEM((EB,), jnp.int32)] * 2,
            [pltpu.VMEM((EB,), jnp.int32)] * 2,
            [pltpu.VMEM((EB, TW), jnp.float32)] * 2,
            [pltpu.VMEM((EB, 2 * HEADS), jnp.float32)] * 2,
            [pltpu.VMEM((EB, TW), jnp.float32)] * 2,
            [pltpu.VMEM((EB,), jnp.int32)] * 2,
            [pltpu.SemaphoreType.DMA] * 2,
            [pltpu.SemaphoreType.DMA] * 2,
            [pltpu.SemaphoreType.DMA] * 2,
            [pltpu.SemaphoreType.DMA] * 2,
            [pltpu.SemaphoreType.DMA] * 2,
        ],
    )
    def edge_kernel(table_hbm, ad_hbm, src_hbm, dst_hbm, zeros_hbm, out_hbm,
                    acc_sh, src_v, dst_v, rows_v, ad_v, msg_v, sdst_v,
                    sem_g, sem_a, sem_s, sem_i, sem_j):
        cid = lax.axis_index("c")
        sid = lax.axis_index("s")
        wid = sid * NC + cid

        zrows = NPAD // NS
        pltpu.sync_copy(zeros_hbm.at[pl.ds(sid * zrows, zrows)],
                        acc_sh.at[pl.ds(sid * zrows, zrows)])
        plsc.subcore_barrier()

        lanes = lax.iota(jnp.int32, 16)
        leq = [lanes == h for h in range(1, HEADS)]

        def fetch_idx(blk, p):
            ebase = wid * chunk + blk * EB
            pltpu.async_copy(src_hbm.at[pl.ds(ebase, EB)], src_v[p], sem_i[p])
            pltpu.async_copy(dst_hbm.at[pl.ds(ebase, EB)], dst_v[p], sem_j[p])

        def wait_idx(p):
            pltpu.make_async_copy(src_hbm.at[pl.ds(0, EB)], src_v[p], sem_i[p]).wait()
            pltpu.make_async_copy(dst_hbm.at[pl.ds(0, EB)], dst_v[p], sem_j[p]).wait()

        def gathers(p):
            pltpu.async_copy(table_hbm.at[src_v[p]], rows_v[p], sem_g[p])
            pltpu.async_copy(ad_hbm.at[dst_v[p]], ad_v[p], sem_a[p])

        def wait_gather(p):
            pltpu.make_async_copy(table_hbm.at[src_v[p]], rows_v[p], sem_g[p]).wait()
            pltpu.make_async_copy(ad_hbm.at[dst_v[p]], ad_v[p], sem_a[p]).wait()

        def wait_scatter(p):
            pltpu.make_async_copy(msg_v[p], acc_sh.at[sdst_v[p]], sem_s[p]).wait()

        def compute(p):
            def group_body(g, c2):
                eidx = lanes + g * 16
                ws = []
                for h in range(HEADS):
                    a_s = plsc.load_gather(
                        rows_v[p], [eidx, jnp.full((16,), DIM + h, jnp.int32)])
                    a_d = plsc.load_gather(
                        ad_v[p], [eidx, jnp.full((16,), HEADS + h, jnp.int32)])
                    al = a_s + a_d
                    al = jnp.where(al >= 0, al, 0.2 * al)
                    ws.append(jnp.exp(al))
                for k in range(16):
                    e = g * 16 + k
                    wbs = [_bcast_lane(w, k) for w in ws]
                    denv = wbs[0]
                    for h in range(1, HEADS):
                        denv = jnp.where(leq[h - 1], wbs[h], denv)
                    for h in range(HEADS):
                        msg_v[p][e, pl.ds(h * DH, 16)] = (
                            rows_v[p][e, pl.ds(h * DH, 16)] * wbs[h])
                    msg_v[p][e, pl.ds(DIM, 16)] = denv
                return c2

            lax.fori_loop(0, EB // 16, group_body, 0)
            pltpu.async_copy(msg_v[p], acc_sh.at[sdst_v[p]], sem_s[p], add=True)

        def sub_body(t, blk, p):
            if p == 0:
                wait_idx(1)
                gathers(1)
            else:
                @pl.when(t < npairs - 1)
                def _():
                    wait_idx(0)
                    gathers(0)
            wait_gather(p)

            @pl.when(t > 0)
            def _():
                wait_scatter(p)
            for c0 in range(0, EB, 16):
                sdst_v[p][pl.ds(c0, 16)] = dst_v[p][pl.ds(c0, 16)]

            @pl.when(t < npairs - 1)
            def _():
                fetch_idx(blk + 2, p)
            compute(p)

        fetch_idx(0, 0)
        fetch_idx(1, 1)
        wait_idx(0)
        gathers(0)

        def pair_body(t, carry):
            sub_body(t, 2 * t, 0)
            sub_body(t, 2 * t + 1, 1)
            return carry

        lax.fori_loop(0, npairs, pair_body, 0)
        wait_scatter(0)
        wait_scatter(1)

        plsc.subcore_barrier()
        pltpu.sync_copy(acc_sh.at[pl.ds(sid * zrows, zrows)],
                        out_hbm.at[cid, pl.ds(sid * zrows, zrows)])

    return edge_kernel


def _att_cat(W, att_src, att_dst):
    a = jnp.zeros((DIM, 2 * HEADS), jnp.float32)
    hs = jnp.arange(DIM) // DH
    ds = jnp.arange(DIM) % DH
    a = a.at[jnp.arange(DIM), hs].set(att_src[hs, ds])
    a = a.at[jnp.arange(DIM), HEADS + hs].set(att_dst[hs, ds])
    return jnp.concatenate([W, W @ a], axis=1)


def kernel(features, edge_index, W1, att_src1, att_dst1, b1, W2, att_src2, att_dst2, b2, m1_w, m1_b, m2_w, m2_b, g1_w, g1_b, g2_w, g2_b, ln_g, ln_b):
    ne = edge_index.shape[1] + N
    epad = ((ne + NW * EB - 1) // (NW * EB)) * (NW * EB)
    loop = jnp.arange(N, dtype=jnp.int32)
    dummy = jnp.full((epad - ne,), N, jnp.int32)
    src = jnp.concatenate([edge_index[0].astype(jnp.int32), loop, dummy])
    dst = jnp.concatenate([edge_index[1].astype(jnp.int32), loop, dummy])
    x = jnp.pad(features, ((0, NPAD - N), (0, 0)))
    zeros_tab = jnp.zeros((NPAD, TW), jnp.float32)

    edge_sc = _sc_edge_kernel(epad)

    wcat1 = _att_cat(W1, att_src1, att_dst1)
    table1, ad1 = _build_table(x, wcat1)
    acc1 = edge_sc(table1, ad1, src, dst, zeros_tab)

    wcat2 = _att_cat(W2, att_src2, att_dst2)
    table2, ad2 = _finalize1(acc1, b1.reshape(1, -1), wcat2)
    acc2 = edge_sc(table2, ad2, src, dst, zeros_tab)

    sh, se = _finalize2(acc2, b2.reshape(1, -1), m1_w, m1_b.reshape(1, -1), m2_w)

    out = _head(sh, se, g1_w, g1_b, g2_w, g2_b, ln_g, ln_b)
    return out.reshape(DIM)
